# baseline (speedup 1.0000x reference)
import functools

import jax
import jax.numpy as jnp
from jax import lax
from jax.experimental import pallas as pl
from jax.experimental.pallas import tpu as pltpu
from jax.experimental.pallas import tpu_sc as plsc

B = 4096
F = 26
VOCAB = 100000
K = 32
ND = 13
AD = 64

NC = 2
NS = 16
NW = NC * NS
R = B * F
RPW = R // NW
CHUNK = 128
NCH = RPW // CHUNK

BLK = 512
NB = B // BLK


def _sc_gather_body(table_hbm, ids_hbm, out_hbm, idx_v, rows_v, sem):
  wid = lax.axis_index("s") * NC + lax.axis_index("c")
  pltpu.sync_copy(ids_hbm.at[wid], idx_v)
  copies = [
      pltpu.async_copy(table_hbm.at[idx_v.at[j]],
                       rows_v.at[pl.ds(j * CHUNK, CHUNK)], sem)
      for j in range(NCH)
  ]
  for c in copies:
    c.wait()
  pltpu.sync_copy(rows_v, out_hbm.at[pl.ds(wid * RPW, RPW)])


def _sc_gather(table, ids2d):
  mesh = plsc.VectorSubcoreMesh(core_axis_name="c", subcore_axis_name="s")
  fn = pl.kernel(
      _sc_gather_body,
      out_type=jax.ShapeDtypeStruct((R, K), jnp.float32),
      scratch_types=[
          pltpu.VMEM((NCH, CHUNK), jnp.int32),
          pltpu.VMEM((RPW, K), jnp.float32),
          pltpu.SemaphoreType.DMA,
      ],
      mesh=mesh,
      compiler_params=pltpu.CompilerParams(use_tc_tiling_on_sc=False),
  )
  return fn(table, ids2d)


def _tc_main_body(g_ref, x_ref, fcW_ref, fcb_ref, ctx_ref, V_ref, acc_ref):
  i = pl.program_id(0)

  @pl.when(i == 0)
  def _():
    acc_ref[...] = jnp.zeros_like(acc_ref)

  fcW = fcW_ref[...]
  fcb = fcb_ref[...]
  ctx = ctx_ref[...]
  num = jnp.zeros((BLK, 1), jnp.float32)
  den = jnp.zeros((BLK, 1), jnp.float32)
  for f in range(F):
    e = g_ref[f]
    h = jnp.tanh(jax.lax.dot(e, fcW, preferred_element_type=jnp.float32) + fcb)
    sc = jnp.sum(h * ctx, axis=1, keepdims=True)
    p = jnp.exp(sc)
    num += p * jnp.sum(e, axis=1, keepdims=True)
    den += p
  s = num / den

  x = x_ref[...]
  xv = jax.lax.dot(x, V_ref[...], preferred_element_type=jnp.float32)
  x2v2 = jax.lax.dot(x * x, V_ref[...] * V_ref[...],
                     preferred_element_type=jnp.float32)
  inter = 0.5 * jnp.sum(xv * xv - x2v2, axis=1, keepdims=True)

  acc_ref[...] += jnp.sum(s + inter, keepdims=True)


def _tc_combine_body(x_ref, linW_ref, linb_ref, acc_ref, y_ref):
  yv = jax.lax.dot_general(linW_ref[...], x_ref[...],
                           (((0,), (1,)), ((), ())),
                           preferred_element_type=jnp.float32)
  y_ref[...] = yv + linb_ref[...] + acc_ref[...] * (1.0 / B)


def kernel(cat_inputs, num_inputs, emb, fc_W, fc_b, context, lin_W, lin_b, V):
  ids = (cat_inputs.astype(jnp.int32).T
         + (jnp.arange(F, dtype=jnp.int32) * VOCAB)[:, None])
  ids2d = ids.reshape(NW, NCH, CHUNK)
  table = emb.reshape(F * VOCAB, K)

  gathered = _sc_gather(table, ids2d)
  g3 = gathered.reshape(F, B, K)

  acc = pl.pallas_call(
      _tc_main_body,
      grid=(NB,),
      in_specs=[
          pl.BlockSpec((F, BLK, K), lambda i: (0, i, 0)),
          pl.BlockSpec((BLK, ND), lambda i: (i, 0)),
          pl.BlockSpec((K, AD), lambda i: (0, 0)),
          pl.BlockSpec((1, AD), lambda i: (0, 0)),
          pl.BlockSpec((1, AD), lambda i: (0, 0)),
          pl.BlockSpec((ND, K), lambda i: (0, 0)),
      ],
      out_specs=pl.BlockSpec((1, 1), lambda i: (0, 0)),
      out_shape=jax.ShapeDtypeStruct((1, 1), jnp.float32),
  )(g3, num_inputs, fc_W, fc_b.reshape(1, AD), context.reshape(1, AD), V)

  y2 = pl.pallas_call(
      _tc_combine_body,
      out_shape=jax.ShapeDtypeStruct((1, B), jnp.float32),
  )(num_inputs, lin_W, lin_b.reshape(1, 1), acc)
  return y2.reshape(B)


# speedup vs baseline: 2.2969x; 2.2969x over previous
import functools

import jax
import jax.numpy as jnp
from jax import lax
from jax.experimental import pallas as pl
from jax.experimental.pallas import tpu as pltpu
from jax.experimental.pallas import tpu_sc as plsc

B = 4096
F = 26
VOCAB = 100000
K = 32
ND = 13
AD = 64

NC = 2
NS = 16
NW = NC * NS
R = B * F
RPW = R // NW
G = (F * VOCAB) // 8

CH = 32
NCHK = RPW // CH

BLK = 512
NB = B // BLK


def _sc_gather_body(table_hbm, ids_hbm, out_hbm,
                    ids_v, slab0, slab1, comp0, comp1,
                    semg0, semg1, semo0, semo1):
  wid = lax.axis_index("s") * NC + lax.axis_index("c")
  base = wid * RPW
  pltpu.sync_copy(ids_hbm.at[wid], ids_v)

  slabs = (slab0, slab1)
  comps = (comp0, comp1)
  semgs = (semg0, semg1)
  semos = (semo0, semo1)

  def issue(c, b):
    for blk in range(CH // 16):
      rv = ids_v[0, pl.ds(c * CH + blk * 16, 16)]
      for t in range(16):
        g = rv[t] >> 3
        pltpu.async_copy(table_hbm.at[g], slabs[b].at[blk * 16 + t], semgs[b])

  def drain_g(b):
    pltpu.make_async_copy(
        table_hbm.at[pl.ds(0, CH)], slabs[b], semgs[b]).wait()

  def compact(c, b):
    for blk in range(CH // 16):
      i_vec = lax.iota(jnp.int32, 16) + 16 * blk
      s_vec = ids_v[0, pl.ds(c * CH + blk * 16, 16)] & 7
      k_vec = jnp.zeros((16,), jnp.int32)
      for k in range(K):
        vals = plsc.load_gather(slabs[b], [i_vec, s_vec, k_vec])
        plsc.store_scatter(comps[b], [i_vec, k_vec], vals)
        k_vec = k_vec + 1

  def write_out(c, b):
    pltpu.async_copy(comps[b], out_hbm.at[pl.ds(base + c * CH, CH)], semos[b])

  def drain_o(b):
    pltpu.make_async_copy(comps[b], out_hbm.at[pl.ds(0, CH)], semos[b]).wait()

  issue(0, 0)

  def step(j, carry):
    for b in range(2):
      c = 2 * j + b

      @pl.when(c + 1 < NCHK)
      def _():
        issue(c + 1, b ^ 1)

      drain_g(b)

      @pl.when(c >= 2)
      def _():
        drain_o(b)

      compact(c, b)
      write_out(c, b)
    return carry

  lax.fori_loop(0, NCHK // 2, step, 0)
  drain_o(0)
  drain_o(1)


def _sc_gather(table3, ids3):
  mesh = plsc.VectorSubcoreMesh(core_axis_name="c", subcore_axis_name="s")
  fn = pl.kernel(
      _sc_gather_body,
      out_type=jax.ShapeDtypeStruct((R, K), jnp.float32),
      scratch_types=[
          pltpu.VMEM((1, RPW), jnp.int32),
          pltpu.VMEM((CH, 8, K), jnp.float32),
          pltpu.VMEM((CH, 8, K), jnp.float32),
          pltpu.VMEM((CH, K), jnp.float32),
          pltpu.VMEM((CH, K), jnp.float32),
          pltpu.SemaphoreType.DMA,
          pltpu.SemaphoreType.DMA,
          pltpu.SemaphoreType.DMA,
          pltpu.SemaphoreType.DMA,
      ],
      mesh=mesh,
      compiler_params=pltpu.CompilerParams(needs_layout_passes=False),
  )
  return fn(table3, ids3)


def _tc_main_body(g_ref, x_ref, fcW_ref, fcb_ref, ctx_ref, V_ref, acc_ref):
  i = pl.program_id(0)

  @pl.when(i == 0)
  def _():
    acc_ref[...] = jnp.zeros_like(acc_ref)

  fcW = fcW_ref[...]
  fcb = fcb_ref[...]
  ctx = ctx_ref[...]
  num = jnp.zeros((BLK, 1), jnp.float32)
  den = jnp.zeros((BLK, 1), jnp.float32)
  for f in range(F):
    e = g_ref[f]
    h = jnp.tanh(jax.lax.dot(e, fcW, preferred_element_type=jnp.float32) + fcb)
    sc = jnp.sum(h * ctx, axis=1, keepdims=True)
    p = jnp.exp(sc)
    num += p * jnp.sum(e, axis=1, keepdims=True)
    den += p
  s = num / den

  x = x_ref[...]
  xv = jax.lax.dot(x, V_ref[...], preferred_element_type=jnp.float32)
  x2v2 = jax.lax.dot(x * x, V_ref[...] * V_ref[...],
                     preferred_element_type=jnp.float32)
  inter = 0.5 * jnp.sum(xv * xv - x2v2, axis=1, keepdims=True)

  acc_ref[...] += jnp.sum(s + inter, keepdims=True)


def _tc_combine_body(x_ref, linW_ref, linb_ref, acc_ref, y_ref):
  yv = jax.lax.dot_general(linW_ref[...], x_ref[...],
                           (((0,), (1,)), ((), ())),
                           preferred_element_type=jnp.float32)
  y_ref[...] = yv + linb_ref[...] + acc_ref[...] * (1.0 / B)


def kernel(cat_inputs, num_inputs, emb, fc_W, fc_b, context, lin_W, lin_b, V):
  ids = (cat_inputs.astype(jnp.int32).T
         + (jnp.arange(F, dtype=jnp.int32) * VOCAB)[:, None])
  ids3 = ids.reshape(NW, 1, RPW)
  table3 = emb.reshape(G, 8, K)

  gathered = _sc_gather(table3, ids3)
  g3 = gathered.reshape(F, B, K)

  acc = pl.pallas_call(
      _tc_main_body,
      grid=(NB,),
      in_specs=[
          pl.BlockSpec((F, BLK, K), lambda i: (0, i, 0)),
          pl.BlockSpec((BLK, ND), lambda i: (i, 0)),
          pl.BlockSpec((K, AD), lambda i: (0, 0)),
          pl.BlockSpec((1, AD), lambda i: (0, 0)),
          pl.BlockSpec((1, AD), lambda i: (0, 0)),
          pl.BlockSpec((ND, K), lambda i: (0, 0)),
      ],
      out_specs=pl.BlockSpec((1, 1), lambda i: (0, 0)),
      out_shape=jax.ShapeDtypeStruct((1, 1), jnp.float32),
  )(g3, num_inputs, fc_W, fc_b.reshape(1, AD), context.reshape(1, AD), V)

  y2 = pl.pallas_call(
      _tc_combine_body,
      out_shape=jax.ShapeDtypeStruct((1, B), jnp.float32),
  )(num_inputs, lin_W, lin_b.reshape(1, 1), acc)
  return y2.reshape(B)
